# baseline (speedup 1.0000x reference)
import functools

import jax
import jax.numpy as jnp
from jax import lax
from jax.experimental import pallas as pl
from jax.experimental.pallas import tpu as pltpu
from jax.experimental.pallas import tpu_sc as plsc

N_ITEMS = 1000000
D = 32
B = 16384
NK = 50
NW = 32
IDXW = 128
UI_ROWS = B // NW
KS_ROWS = (B * NK) // NW
CHUNK = 512
N_CHUNK = KS_ROWS // CHUNK
RPC = CHUNK // IDXW


def _sc_gather(i2, ks2, table):
    mesh = plsc.VectorSubcoreMesh(core_axis_name="c", subcore_axis_name="s")

    @functools.partial(
        pl.kernel,
        mesh=mesh,
        out_type=(
            jax.ShapeDtypeStruct((B, D), jnp.float32),
            jax.ShapeDtypeStruct((B * NK, D), jnp.float32),
        ),
        scratch_types=[
            pltpu.VMEM((RPC, IDXW), jnp.int32),
            pltpu.VMEM((CHUNK, D), jnp.float32),
            pltpu.SemaphoreType.DMA,
        ],
        compiler_params=pltpu.CompilerParams(use_tc_tiling_on_sc=False),
    )
    def k(i2_hbm, ks2_hbm, table_hbm, out_ui, out_uks, idx_v, rows_v, sem):
        wid = lax.axis_index("s") * 2 + lax.axis_index("c")

        pltpu.sync_copy(i2_hbm.at[pl.ds(wid * RPC, RPC)], idx_v)
        cps = [
            pltpu.async_copy(
                table_hbm.at[idx_v.at[r]],
                rows_v.at[pl.ds(r * IDXW, IDXW)],
                sem,
            )
            for r in range(RPC)
        ]
        for c in cps:
            c.wait()
        pltpu.sync_copy(rows_v, out_ui.at[pl.ds(wid * UI_ROWS, UI_ROWS)])

        def body(c, carry):
            irow = wid * (KS_ROWS // IDXW) + c * RPC
            pltpu.sync_copy(ks2_hbm.at[pl.ds(irow, RPC)], idx_v)
            cps = [
                pltpu.async_copy(
                    table_hbm.at[idx_v.at[r]],
                    rows_v.at[pl.ds(r * IDXW, IDXW)],
                    sem,
                )
                for r in range(RPC)
            ]
            for cp in cps:
                cp.wait()
            pltpu.sync_copy(
                rows_v, out_uks.at[pl.ds(wid * KS_ROWS + c * CHUNK, CHUNK)]
            )
            return carry

        lax.fori_loop(0, N_CHUNK, body, 0)

    return k(i2, ks2, table)


BS = 256


def _tc_body(ui_ref, uks_ref, out_ref):
    ui = ui_ref[...]
    uks = uks_ref[...].reshape(BS, NK, D)
    m = ui[:, None, :] * uks
    tot = jnp.sum(m, axis=-1)
    d = 2.0 * m[:, :, 0] - tot
    d = jnp.maximum(d, 1.0)
    dist = -jnp.log(d + jnp.sqrt(d * d - 1.0))
    lse = jnp.log(jnp.sum(jnp.exp(dist), axis=-1) + 1e-6)
    out_ref[...] = (lse - dist[:, 0])[None, None, :]


def _tc_loss(ui, uks):
    grid = B // BS
    out = pl.pallas_call(
        _tc_body,
        grid=(grid,),
        in_specs=[
            pl.BlockSpec((BS, D), lambda i: (i, 0)),
            pl.BlockSpec((BS * NK, D), lambda i: (i, 0)),
        ],
        out_specs=pl.BlockSpec((1, 1, BS), lambda i: (i, 0, 0)),
        out_shape=jax.ShapeDtypeStruct((grid, 1, BS), jnp.float32),
    )(ui, uks)
    return out.reshape(B)


def kernel(I, Ks, table):
    i2 = I.astype(jnp.int32).reshape(B // IDXW, IDXW)
    ks2 = Ks.astype(jnp.int32).reshape((B * NK) // IDXW, IDXW)
    ui, uks = _sc_gather(i2, ks2, table)
    loss = _tc_loss(ui, uks)
    return (loss, table)


# speedup vs baseline: 1.3438x; 1.3438x over previous
import functools

import jax
import jax.numpy as jnp
from jax import lax
from jax.experimental import pallas as pl
from jax.experimental.pallas import tpu as pltpu
from jax.experimental.pallas import tpu_sc as plsc

N_ITEMS = 1000000
D = 32
B = 16384
NK = 50
NW = 32
IDXW = 128
UI_ROWS = B // NW
KS_ROWS = (B * NK) // NW
KS_IROWS = KS_ROWS // IDXW
CHUNK = 512
N_CHUNK = KS_ROWS // CHUNK
RPC = CHUNK // IDXW
GROUPS = CHUNK // 16


def _sc_dists(i2, ks2, table):
    mesh = plsc.VectorSubcoreMesh(core_axis_name="c", subcore_axis_name="s")

    @functools.partial(
        pl.kernel,
        mesh=mesh,
        out_type=jax.ShapeDtypeStruct((B * NK,), jnp.float32),
        scratch_types=[
            pltpu.VMEM((UI_ROWS, D), jnp.float32),
            pltpu.VMEM((KS_IROWS, IDXW), jnp.int32),
            pltpu.VMEM((CHUNK, D), jnp.float32),
            pltpu.VMEM((CHUNK, D), jnp.float32),
            pltpu.VMEM((KS_ROWS,), jnp.float32),
            pltpu.SemaphoreType.DMA,
            pltpu.SemaphoreType.DMA,
            pltpu.SemaphoreType.DMA,
        ],
        compiler_params=pltpu.CompilerParams(
            use_tc_tiling_on_sc=False, needs_layout_passes=False
        ),
    )
    def k(i2_hbm, ks2_hbm, table_hbm, out_d, ui_v, idx_v, rows0, rows1,
          d_v, semu, sem0, sem1):
        wid = lax.axis_index("s") * 2 + lax.axis_index("c")
        lanes = lax.iota(jnp.int32, 16)
        zeros16 = jnp.zeros((16,), jnp.int32)

        pltpu.sync_copy(ks2_hbm.at[pl.ds(wid * KS_IROWS, KS_IROWS)], idx_v)
        pltpu.sync_copy(
            i2_hbm.at[pl.ds(wid * (UI_ROWS // IDXW), UI_ROWS // IDXW)],
            idx_v.at[pl.ds(0, UI_ROWS // IDXW)],
        )
        uicps = [
            pltpu.async_copy(
                table_hbm.at[idx_v.at[r]],
                ui_v.at[pl.ds(r * IDXW, IDXW)],
                semu,
            )
            for r in range(UI_ROWS // IDXW)
        ]
        for cp in uicps:
            cp.wait()
        pltpu.sync_copy(
            ks2_hbm.at[pl.ds(wid * KS_IROWS, UI_ROWS // IDXW)],
            idx_v.at[pl.ds(0, UI_ROWS // IDXW)],
        )
        def neg_body(r, carry):
            rvec = r * 16 + lanes
            c0 = plsc.load_gather(ui_v, [rvec, zeros16])
            plsc.store_scatter(ui_v, [rvec, zeros16], -c0)
            return carry

        lax.fori_loop(0, UI_ROWS // 16, neg_body, 0)

        def fire(c, rows, sem):
            for r in range(RPC):
                pltpu.async_copy(
                    table_hbm.at[idx_v.at[c * RPC + r]],
                    rows.at[pl.ds(r * IDXW, IDXW)],
                    sem,
                )

        def drain(rows, sem):
            pltpu.make_async_copy(
                table_hbm.at[pl.ds(0, CHUNK)], rows, sem
            ).wait()

        def compute(c, rows):
            def g_body(g, carry):
                lp = g * 16 + lanes
                bloc = (c * CHUNK + g * 16 + lanes) // NK
                acc = jnp.zeros((16,), jnp.float32)
                for j in range(D):
                    jv = jnp.full((16,), j, jnp.int32)
                    kv = plsc.load_gather(rows, [lp, jv])
                    uv = plsc.load_gather(ui_v, [bloc, jv])
                    acc = acc + kv * uv
                d_v[pl.ds(c * CHUNK + g * 16, 16)] = -acc
                return carry

            lax.fori_loop(0, GROUPS, g_body, 0)

        fire(0, rows0, sem0)

        def pair_body(i, carry):
            fire(2 * i + 1, rows1, sem1)
            drain(rows0, sem0)
            compute(2 * i, rows0)

            @pl.when(i < N_CHUNK // 2 - 1)
            def _():
                fire(2 * i + 2, rows0, sem0)

            drain(rows1, sem1)
            compute(2 * i + 1, rows1)
            return carry

        lax.fori_loop(0, N_CHUNK // 2, pair_body, 0)
        pltpu.sync_copy(d_v, out_d.at[pl.ds(wid * KS_ROWS, KS_ROWS)])

    return k(i2, ks2, table)


TBS = 2048


def _tc_body(d_ref, out_ref):
    d = jnp.maximum(d_ref[...], 1.0)
    dist = -jnp.log(d + jnp.sqrt(d * d - 1.0))
    lse = jnp.log(jnp.sum(jnp.exp(dist), axis=-1) + 1e-6)
    out_ref[...] = (lse - dist[:, 0])[None, None, :]


def _tc_loss(draw):
    grid = B // TBS
    out = pl.pallas_call(
        _tc_body,
        grid=(grid,),
        in_specs=[pl.BlockSpec((TBS, NK), lambda i: (i, 0))],
        out_specs=pl.BlockSpec((1, 1, TBS), lambda i: (i, 0, 0)),
        out_shape=jax.ShapeDtypeStruct((grid, 1, TBS), jnp.float32),
    )(draw)
    return out.reshape(B)


def kernel(I, Ks, table):
    i2 = I.astype(jnp.int32).reshape(B // IDXW, IDXW)
    ks2 = Ks.astype(jnp.int32).reshape((B * NK) // IDXW, IDXW)
    draw = _sc_dists(i2, ks2, table).reshape(B, NK)
    loss = _tc_loss(draw)
    return (loss, table)


# speedup vs baseline: 2.1324x; 1.5868x over previous
import functools

import jax
import jax.numpy as jnp
from jax import lax
from jax.experimental import pallas as pl
from jax.experimental.pallas import tpu as pltpu
from jax.experimental.pallas import tpu_sc as plsc

N_ITEMS = 1000000
D = 32
B = 16384
NK = 50
NW = 32
IDXW = 128
UI_ROWS = B // NW
KS_ROWS = (B * NK) // NW
KS_IROWS = KS_ROWS // IDXW
CHUNK = 512
N_CHUNK = KS_ROWS // CHUNK
RPC = CHUNK // IDXW
GROUPS = CHUNK // 16


def _sc_dists(i2, ks2, table):
    mesh = plsc.VectorSubcoreMesh(core_axis_name="c", subcore_axis_name="s")

    @functools.partial(
        pl.kernel,
        mesh=mesh,
        out_type=jax.ShapeDtypeStruct((B * NK,), jnp.float32),
        scratch_types=[
            pltpu.VMEM((UI_ROWS, D), jnp.float32),
            pltpu.VMEM((KS_IROWS, IDXW), jnp.int32),
            pltpu.VMEM((CHUNK, D), jnp.float32),
            pltpu.VMEM((CHUNK, D), jnp.float32),
            pltpu.VMEM((KS_ROWS,), jnp.float32),
            pltpu.SemaphoreType.DMA,
            pltpu.SemaphoreType.DMA,
            pltpu.SemaphoreType.DMA,
        ],
        compiler_params=pltpu.CompilerParams(
            use_tc_tiling_on_sc=False, needs_layout_passes=False
        ),
    )
    def k(i2_hbm, ks2_hbm, table_hbm, out_d, ui_v, idx_v, rows0, rows1,
          d_v, semu, sem0, sem1):
        wid = lax.axis_index("s") * 2 + lax.axis_index("c")
        lanes = lax.iota(jnp.int32, 16)
        zeros16 = jnp.zeros((16,), jnp.int32)

        pltpu.sync_copy(ks2_hbm.at[pl.ds(wid * KS_IROWS, KS_IROWS)], idx_v)
        pltpu.sync_copy(
            i2_hbm.at[pl.ds(wid * (UI_ROWS // IDXW), UI_ROWS // IDXW)],
            idx_v.at[pl.ds(0, UI_ROWS // IDXW)],
        )
        uicps = [
            pltpu.async_copy(
                table_hbm.at[idx_v.at[r]],
                ui_v.at[pl.ds(r * IDXW, IDXW)],
                semu,
            )
            for r in range(UI_ROWS // IDXW)
        ]
        for cp in uicps:
            cp.wait()
        pltpu.sync_copy(
            ks2_hbm.at[pl.ds(wid * KS_IROWS, UI_ROWS // IDXW)],
            idx_v.at[pl.ds(0, UI_ROWS // IDXW)],
        )
        def neg_body(r, carry):
            rvec = r * 16 + lanes
            c0 = plsc.load_gather(ui_v, [rvec, zeros16])
            plsc.store_scatter(ui_v, [rvec, zeros16], -c0)
            return carry

        lax.fori_loop(0, UI_ROWS // 16, neg_body, 0)

        def fire(c, rows, sem):
            for r in range(RPC):
                pltpu.async_copy(
                    table_hbm.at[idx_v.at[c * RPC + r]],
                    rows.at[pl.ds(r * IDXW, IDXW)],
                    sem,
                )

        def drain(rows, sem):
            pltpu.make_async_copy(
                table_hbm.at[pl.ds(0, CHUNK)], rows, sem
            ).wait()

        def compute(c, rows):
            def g_body(g, carry):
                bvec = (c * CHUNK + g * 16 + lanes) // NK
                dvec = jnp.zeros((16,), jnp.float32)
                for t in range(16):
                    p = g * 16 + t
                    b = bvec[t]
                    u0 = ui_v[b, 0:16]
                    u1 = ui_v[b, 16:32]
                    k0 = rows[p, 0:16]
                    k1 = rows[p, 16:32]
                    s = jnp.sum(k0 * u0 + k1 * u1)
                    dvec = jnp.where(lanes == t, s, dvec)
                d_v[pl.ds(c * CHUNK + g * 16, 16)] = -dvec
                return carry

            lax.fori_loop(0, GROUPS, g_body, 0)

        fire(0, rows0, sem0)

        def pair_body(i, carry):
            fire(2 * i + 1, rows1, sem1)
            drain(rows0, sem0)
            compute(2 * i, rows0)

            @pl.when(i < N_CHUNK // 2 - 1)
            def _():
                fire(2 * i + 2, rows0, sem0)

            drain(rows1, sem1)
            compute(2 * i + 1, rows1)
            return carry

        lax.fori_loop(0, N_CHUNK // 2, pair_body, 0)
        pltpu.sync_copy(d_v, out_d.at[pl.ds(wid * KS_ROWS, KS_ROWS)])

    return k(i2, ks2, table)


TBS = 2048


def _tc_body(d_ref, out_ref):
    d = jnp.maximum(d_ref[...], 1.0)
    dist = -jnp.log(d + jnp.sqrt(d * d - 1.0))
    lse = jnp.log(jnp.sum(jnp.exp(dist), axis=-1) + 1e-6)
    out_ref[...] = (lse - dist[:, 0])[None, None, :]


def _tc_loss(draw):
    grid = B // TBS
    out = pl.pallas_call(
        _tc_body,
        grid=(grid,),
        in_specs=[pl.BlockSpec((TBS, NK), lambda i: (i, 0))],
        out_specs=pl.BlockSpec((1, 1, TBS), lambda i: (i, 0, 0)),
        out_shape=jax.ShapeDtypeStruct((grid, 1, TBS), jnp.float32),
    )(draw)
    return out.reshape(B)


def kernel(I, Ks, table):
    i2 = I.astype(jnp.int32).reshape(B // IDXW, IDXW)
    ks2 = Ks.astype(jnp.int32).reshape((B * NK) // IDXW, IDXW)
    draw = _sc_dists(i2, ks2, table).reshape(B, NK)
    loss = _tc_loss(draw)
    return (loss, table)
